# baseline (speedup 1.0000x reference)
import functools

import jax
import jax.numpy as jnp
from jax import lax
from jax.experimental import pallas as pl
from jax.experimental.pallas import tpu as pltpu
from jax.experimental.pallas import tpu_sc as plsc

N = 10000
E = 320000
D = 128

NC = 2
NS = 16
NW = NC * NS

B = 128
EPT = E // NW
NBUF = 2
NBLK = 80
EPT_PAD = NBLK * B
NPAD = 10240
RPT = NPAD // NS
HR = NPAD // D

SB = 14
SMASK = (1 << SB) - 1

_mesh = plsc.VectorSubcoreMesh(
    core_axis_name="c", subcore_axis_name="s", num_cores=NC, num_subcores=NS
)


@functools.partial(
    pl.kernel,
    out_type=jax.ShapeDtypeStruct((NC, NPAD, D), jnp.float32),
    mesh=_mesh,
    compiler_params=pltpu.CompilerParams(needs_layout_passes=False),
    scratch_types=[
        pltpu.VMEM((NBLK, B), jnp.int32),
        pltpu.VMEM((B, D), jnp.float32),
        pltpu.VMEM((B, D), jnp.float32),
        pltpu.VMEM((B,), jnp.int32),
        pltpu.VMEM((B,), jnp.int32),
        pltpu.VMEM((B,), jnp.int32),
        pltpu.VMEM_SHARED((NPAD, D), jnp.float32),
        pltpu.SemaphoreType.DMA,
        pltpu.SemaphoreType.DMA,
    ],
)
def _sc_agg(h_hbm, pk_hbm, out_hbm, pk_v, r0, r1, si0, si1, di_v, acc_sh,
            sem0, sem1):
    c = lax.axis_index("c")
    s = lax.axis_index("s")
    rows = (r0, r1)
    sibuf = (si0, si1)
    sems = (sem0, sem1)

    pltpu.sync_copy(pk_hbm.at[c, s], pk_v)

    z = jnp.zeros((16,), jnp.float32)

    @pl.loop(0, B)
    def _(r):
        for k in range(D // 16):
            r0[r, pl.ds(k * 16, 16)] = z

    for k in range(RPT // B):
        pltpu.sync_copy(r0, acc_sh.at[pl.ds(s * RPT + k * B, B)])
    plsc.subcore_barrier()

    def unpack_src(blk, buf):
        for k in range(B // 16):
            pk = pk_v[blk, pl.ds(k * 16, 16)]
            buf[pl.ds(k * 16, 16)] = lax.bitwise_and(pk, SMASK)

    def unpack_dst(blk):
        for k in range(B // 16):
            pk = pk_v[blk, pl.ds(k * 16, 16)]
            di_v[pl.ds(k * 16, 16)] = lax.shift_right_logical(pk, SB)

    for b in range(NBUF):
        unpack_src(b, sibuf[b])
        pltpu.async_copy(h_hbm.at[sibuf[b]], rows[b], sems[b])

    @pl.loop(0, NBLK, step=NBUF)
    def _(j):
        for b in range(NBUF):
            blk = j + b
            pltpu.make_async_copy(
                h_hbm.at[sibuf[b]], rows[b], sems[b]
            ).wait()
            unpack_dst(blk)
            pltpu.sync_copy(rows[b], acc_sh.at[di_v], add=True)

            @pl.when(blk + NBUF < NBLK)
            def _():
                unpack_src(blk + NBUF, sibuf[b])
                pltpu.async_copy(h_hbm.at[sibuf[b]], rows[b], sems[b])

    plsc.subcore_barrier()
    pltpu.sync_copy(
        acc_sh.at[pl.ds(s * RPT, RPT)], out_hbm.at[c, pl.ds(s * RPT, RPT)]
    )


@functools.partial(
    pl.kernel,
    out_type=jax.ShapeDtypeStruct((NW, NPAD), jnp.float32),
    mesh=_mesh,
    compiler_params=pltpu.CompilerParams(needs_layout_passes=False),
    scratch_types=[
        pltpu.VMEM((NBLK, B), jnp.int32),
        pltpu.VMEM((NPAD,), jnp.float32),
    ],
)
def _sc_deg(dst_hbm, out_hbm, dst_v, hist_v):
    c = lax.axis_index("c")
    s = lax.axis_index("s")

    pltpu.sync_copy(dst_hbm.at[c, s], dst_v)

    z = jnp.zeros((16,), jnp.float32)

    @pl.loop(0, NPAD // 16)
    def _(i):
        hist_v[pl.ds(i * 16, 16)] = z

    one = jnp.ones((16,), jnp.float32)

    @pl.loop(0, NBLK)
    def _(j):
        for k in range(B // 16):
            idx = dst_v[j, pl.ds(k * 16, 16)]
            plsc.addupdate_scatter(hist_v, [idx], one)

    w = c * NS + s
    pltpu.sync_copy(hist_v, out_hbm.at[w])


TC_BLK = 1280
HB = TC_BLK // D


def _tc_post_body(p_ref, dp_ref, w_ref, b_ref, o_ref):
    agg = p_ref[0] + p_ref[1]
    deg = jnp.sum(dp_ref[...], axis=1)
    deg = jnp.maximum(deg, 1.0)
    h = (agg.reshape(HB, D, D) / deg[:, :, None]).reshape(TC_BLK, D)
    acc = jnp.dot(h, w_ref[...], preferred_element_type=jnp.float32)
    o_ref[...] = jnp.maximum(acc + b_ref[...], 0.0)


def _tc_post(p, degt, w, b):
    return pl.pallas_call(
        _tc_post_body,
        grid=(NPAD // TC_BLK,),
        in_specs=[
            pl.BlockSpec((NC, TC_BLK, D), lambda i: (0, i, 0)),
            pl.BlockSpec((HB, NW, D), lambda i: (i, 0, 0)),
            pl.BlockSpec((D, D), lambda i: (0, 0)),
            pl.BlockSpec((1, D), lambda i: (0, 0)),
        ],
        out_specs=pl.BlockSpec((TC_BLK, D), lambda i: (i, 0)),
        out_shape=jax.ShapeDtypeStruct((NPAD, D), jnp.float32),
    )(p, degt, w, b.reshape(1, D))


def kernel(x, edge_index, W1, b1, W2, b2, W3, b3):
    src = edge_index[0]
    dst = edge_index[1]
    srcb = jnp.pad(src.reshape(NW, EPT), ((0, 0), (0, EPT_PAD - EPT)))
    dstb = jnp.pad(
        dst.reshape(NW, EPT), ((0, 0), (0, EPT_PAD - EPT)), constant_values=N
    )
    pkb = (srcb | (dstb << SB)).reshape(NC, NS, NBLK, B)
    dstb = dstb.reshape(NC, NS, NBLK, B)

    degh = _sc_deg(dstb)
    degt = degh.reshape(NW, HR, D).transpose(1, 0, 2)

    h = jnp.pad(x, ((0, NPAD - N), (0, 0)))
    for w, b in ((W1, b1), (W2, b2), (W3, b3)):
        p = _sc_agg(h, pkb)
        h = _tc_post(p, degt, w, b)
    return h[:N]
